# baseline (speedup 1.0000x reference)
import functools

import jax
import jax.numpy as jnp
from jax import lax
from jax.experimental import pallas as pl
from jax.experimental.pallas import tpu as pltpu
from jax.experimental.pallas import tpu_sc as plsc

_NC = 2
_NS = 16
_NW = _NC * _NS
_T = 128
_S0 = 80
_S1 = 80


def _sc_aggregate(feat, src2, dst2, n_pad):
    n, d = feat.shape
    _, t = src2.shape
    smax = max(_S0, _S1)
    rows_per_tile = n_pad // _NS

    mesh = plsc.VectorSubcoreMesh(core_axis_name="c", subcore_axis_name="s")

    @functools.partial(
        pl.kernel,
        mesh=mesh,
        out_type=jax.ShapeDtypeStruct((_NC, n_pad, d), jnp.float32),
        scratch_types=[
            pltpu.VMEM((smax, t), jnp.int32),
            pltpu.VMEM((smax, t), jnp.int32),
            pltpu.VMEM((t, d), jnp.float32),
            pltpu.VMEM_SHARED((n_pad, d), jnp.float32),
            pltpu.SemaphoreType.DMA,
        ],
    )
    def agg_kernel(feat_hbm, src_hbm, dst_hbm, out_hbm,
                   src_v, dst_v, rows_v, agg_sh, sem):
        c = lax.axis_index("c")
        s = lax.axis_index("s")

        vecs_per_row = d // 16

        def zero_body(i, carry):
            r = i // vecs_per_row
            col = (i % vecs_per_row) * 16
            rows_v[r, pl.ds(col, 16)] = jnp.zeros((16,), jnp.float32)
            return carry

        lax.fori_loop(0, t * vecs_per_row, zero_body, 0)
        off = 0
        while off < rows_per_tile:
            cz = min(t, rows_per_tile - off)
            zsrc = rows_v if cz == t else rows_v.at[pl.ds(0, cz)]
            pltpu.sync_copy(zsrc,
                            agg_sh.at[pl.ds(s * rows_per_tile + off, cz)])
            off += cz
        plsc.subcore_barrier()

        base = jnp.where(c == 0, s * _S0, _NS * _S0 + s * _S1)

        def run(nsteps):
            if nsteps == 0:
                return
            sslice = src_v if nsteps == smax else src_v.at[pl.ds(0, nsteps)]
            dslice = dst_v if nsteps == smax else dst_v.at[pl.ds(0, nsteps)]
            pltpu.sync_copy(src_hbm.at[pl.ds(base, nsteps)], sslice)
            pltpu.sync_copy(dst_hbm.at[pl.ds(base, nsteps)], dslice)

            def body(j, carry):
                pltpu.async_copy(feat_hbm.at[src_v.at[j]], rows_v, sem).wait()
                pltpu.sync_copy(rows_v, agg_sh.at[dst_v.at[j]], add=True)
                return carry

            lax.fori_loop(0, nsteps, body, 0)

        pl.when(c == 0)(lambda: run(_S0))
        pl.when(c == 1)(lambda: run(_S1))

        plsc.subcore_barrier()
        pltpu.sync_copy(agg_sh.at[pl.ds(s * rows_per_tile, rows_per_tile)],
                        out_hbm.at[c, pl.ds(s * rows_per_tile, rows_per_tile)])

    return agg_kernel(feat, src2, dst2)


def _tc_finish(partials, feat, deg, wt1, wt2):
    n, d = feat.shape
    r = 1000

    def body(p_ref, f_ref, deg_ref, w1_ref, w2_ref, o_ref):
        agg = p_ref[0] + p_ref[1]
        inv = 1.0 / jnp.maximum(deg_ref[...], 1.0)
        h = jnp.dot(agg * inv, w1_ref[...], preferred_element_type=jnp.float32)
        h = h + jnp.dot(f_ref[...], w2_ref[...],
                        preferred_element_type=jnp.float32)
        h = jnp.maximum(h, 0.0)
        nrm = jnp.sqrt(jnp.sum(h * h, axis=-1, keepdims=True))
        o_ref[...] = h / jnp.maximum(nrm, 1e-12)

    return pl.pallas_call(
        body,
        grid=(n // r,),
        in_specs=[
            pl.BlockSpec((_NC, r, d), lambda i: (0, i, 0)),
            pl.BlockSpec((r, d), lambda i: (i, 0)),
            pl.BlockSpec((r, 1), lambda i: (i, 0)),
            pl.BlockSpec((d, d), lambda i: (0, 0)),
            pl.BlockSpec((d, d), lambda i: (0, 0)),
        ],
        out_specs=pl.BlockSpec((r, d), lambda i: (i, 0)),
        out_shape=jax.ShapeDtypeStruct((n, d), jnp.float32),
    )(partials, feat, deg, wt1, wt2)


def kernel(feat, edge, degree, W):
    n, d = feat.shape
    e = edge.shape[0]

    n_pad = -(-n // (_NS * 8)) * (_NS * 8)
    e_pad = _NS * (_S0 + _S1) * _T
    assert e_pad >= e, (e_pad, e)
    pad = e_pad - e
    src = jnp.concatenate([edge[:, 0], jnp.zeros((pad,), jnp.int32)])
    dst = jnp.concatenate([edge[:, 1], jnp.full((pad,), n, jnp.int32)])
    src2 = src.reshape(e_pad // _T, _T)
    dst2 = dst.reshape(e_pad // _T, _T)

    partials = _sc_aggregate(feat, src2, dst2, n_pad)

    wt = W.T
    deg2 = degree.astype(jnp.float32).reshape(n, 1)
    return _tc_finish(partials, feat, deg2, wt[:d], wt[d:])


# speedup vs baseline: 1.5605x; 1.5605x over previous
import functools

import jax
import jax.numpy as jnp
from jax import lax
from jax.experimental import pallas as pl
from jax.experimental.pallas import tpu as pltpu
from jax.experimental.pallas import tpu_sc as plsc

_NC = 2
_NS = 16
_NW = _NC * _NS
_T = 128
def _sc_aggregate(feat, src3, dst3, n_pad):
    n, d = feat.shape
    _, steps, t = src3.shape
    rows_per_tile = n_pad // _NS

    mesh = plsc.VectorSubcoreMesh(core_axis_name="c", subcore_axis_name="s")

    @functools.partial(
        pl.kernel,
        mesh=mesh,
        out_type=jax.ShapeDtypeStruct((_NC, n_pad, d), jnp.float32),
        scratch_types=[
            pltpu.VMEM((steps, t), jnp.int32),
            pltpu.VMEM((steps, t), jnp.int32),
            pltpu.VMEM((t, d), jnp.float32),
            pltpu.VMEM_SHARED((n_pad, d), jnp.float32),
            pltpu.SemaphoreType.DMA,
        ],
    )
    def agg_kernel(feat_hbm, src_hbm, dst_hbm, out_hbm,
                   src_v, dst_v, rows_v, agg_sh, sem):
        c = lax.axis_index("c")
        s = lax.axis_index("s")
        tid = c * _NS + s

        vecs_per_row = d // 16

        def zero_body(i, carry):
            r = i // vecs_per_row
            col = (i % vecs_per_row) * 16
            rows_v[r, pl.ds(col, 16)] = jnp.zeros((16,), jnp.float32)
            return carry

        lax.fori_loop(0, t * vecs_per_row, zero_body, 0)
        for k in range(rows_per_tile // t):
            pltpu.sync_copy(rows_v,
                            agg_sh.at[pl.ds(s * rows_per_tile + k * t, t)])
        plsc.subcore_barrier()

        pltpu.sync_copy(src_hbm.at[tid], src_v)
        pltpu.sync_copy(dst_hbm.at[tid], dst_v)

        def body(j, carry):
            pltpu.async_copy(feat_hbm.at[src_v.at[j]], rows_v, sem).wait()
            pltpu.sync_copy(rows_v, agg_sh.at[dst_v.at[j]], add=True)
            return carry

        lax.fori_loop(0, steps, body, 0)

        plsc.subcore_barrier()
        pltpu.sync_copy(agg_sh.at[pl.ds(s * rows_per_tile, rows_per_tile)],
                        out_hbm.at[c, pl.ds(s * rows_per_tile, rows_per_tile)])

    return agg_kernel(feat, src3, dst3)


def _tc_finish(partials, feat, deg, wt1, wt2):
    n, d = feat.shape
    r = 1000

    def body(p_ref, f_ref, deg_ref, w1_ref, w2_ref, o_ref):
        agg = p_ref[0] + p_ref[1]
        inv = 1.0 / jnp.maximum(deg_ref[...], 1.0)
        h = jnp.dot(agg * inv, w1_ref[...], preferred_element_type=jnp.float32)
        h = h + jnp.dot(f_ref[...], w2_ref[...],
                        preferred_element_type=jnp.float32)
        h = jnp.maximum(h, 0.0)
        nrm = jnp.sqrt(jnp.sum(h * h, axis=-1, keepdims=True))
        o_ref[...] = h / jnp.maximum(nrm, 1e-12)

    return pl.pallas_call(
        body,
        grid=(n // r,),
        in_specs=[
            pl.BlockSpec((_NC, r, d), lambda i: (0, i, 0)),
            pl.BlockSpec((r, d), lambda i: (i, 0)),
            pl.BlockSpec((r, 1), lambda i: (i, 0)),
            pl.BlockSpec((d, d), lambda i: (0, 0)),
            pl.BlockSpec((d, d), lambda i: (0, 0)),
        ],
        out_specs=pl.BlockSpec((r, d), lambda i: (i, 0)),
        out_shape=jax.ShapeDtypeStruct((n, d), jnp.float32),
    )(partials, feat, deg, wt1, wt2)


def kernel(feat, edge, degree, W):
    n, d = feat.shape
    e = edge.shape[0]

    n_pad = -(-n // (_NS * _T)) * (_NS * _T)
    e_pad = -(-e // (_NW * _T)) * (_NW * _T)
    pad = e_pad - e
    src = jnp.concatenate([edge[:, 0], jnp.zeros((pad,), jnp.int32)])
    dst = jnp.concatenate([edge[:, 1], jnp.full((pad,), n, jnp.int32)])
    steps = e_pad // (_NW * _T)
    src3 = src.reshape(_NW, steps, _T)
    dst3 = dst.reshape(_NW, steps, _T)

    partials = _sc_aggregate(feat, src3, dst3, n_pad)

    wt = W.T
    deg2 = degree.astype(jnp.float32).reshape(n, 1)
    return _tc_finish(partials, feat, deg2, wt[:d], wt[d:])


# speedup vs baseline: 1.5616x; 1.0007x over previous
import functools

import jax
import jax.numpy as jnp
from jax import lax
from jax.experimental import pallas as pl
from jax.experimental.pallas import tpu as pltpu
from jax.experimental.pallas import tpu_sc as plsc

_NC = 2
_NS = 16
_NW = _NC * _NS
_T = 128
def _sc_aggregate(feat, src3, dst3, n_pad):
    n, d = feat.shape
    _, steps, t = src3.shape
    rows_per_tile = n_pad // _NS

    mesh = plsc.VectorSubcoreMesh(core_axis_name="c", subcore_axis_name="s")

    @functools.partial(
        pl.kernel,
        mesh=mesh,
        out_type=jax.ShapeDtypeStruct((_NC, n_pad, d), jnp.float32),
        scratch_types=[
            pltpu.VMEM((steps, t), jnp.int32),
            pltpu.VMEM((steps, t), jnp.int32),
            pltpu.VMEM((t, d), jnp.float32),
            pltpu.VMEM_SHARED((n_pad, d), jnp.float32),
            pltpu.SemaphoreType.DMA,
        ],
    )
    def agg_kernel(feat_hbm, src_hbm, dst_hbm, out_hbm,
                   src_v, dst_v, rows_v, agg_sh, sem):
        c = lax.axis_index("c")
        s = lax.axis_index("s")
        tid = c * _NS + s

        vecs_per_row = d // 16

        def zero_body(i, carry):
            r = i // vecs_per_row
            col = (i % vecs_per_row) * 16
            rows_v[r, pl.ds(col, 16)] = jnp.zeros((16,), jnp.float32)
            return carry

        lax.fori_loop(0, t * vecs_per_row, zero_body, 0)
        off = 0
        while off < rows_per_tile:
            cz = min(t, rows_per_tile - off)
            zsrc = rows_v if cz == t else rows_v.at[pl.ds(0, cz)]
            pltpu.sync_copy(zsrc,
                            agg_sh.at[pl.ds(s * rows_per_tile + off, cz)])
            off += cz
        plsc.subcore_barrier()

        pltpu.sync_copy(src_hbm.at[tid], src_v)
        pltpu.sync_copy(dst_hbm.at[tid], dst_v)

        def body(j, carry):
            pltpu.async_copy(feat_hbm.at[src_v.at[j]], rows_v, sem).wait()
            pltpu.sync_copy(rows_v, agg_sh.at[dst_v.at[j]], add=True)
            return carry

        lax.fori_loop(0, steps, body, 0)

        plsc.subcore_barrier()
        pltpu.sync_copy(agg_sh.at[pl.ds(s * rows_per_tile, rows_per_tile)],
                        out_hbm.at[c, pl.ds(s * rows_per_tile, rows_per_tile)])

    return agg_kernel(feat, src3, dst3)


def _tc_finish(partials, feat, deg, wt1, wt2):
    n, d = feat.shape
    r = 1000

    def body(p_ref, f_ref, deg_ref, w1_ref, w2_ref, o_ref):
        agg = p_ref[0] + p_ref[1]
        inv = 1.0 / jnp.maximum(deg_ref[...], 1.0)
        h = jnp.dot(agg * inv, w1_ref[...], preferred_element_type=jnp.float32)
        h = h + jnp.dot(f_ref[...], w2_ref[...],
                        preferred_element_type=jnp.float32)
        h = jnp.maximum(h, 0.0)
        nrm = jnp.sqrt(jnp.sum(h * h, axis=-1, keepdims=True))
        o_ref[...] = h / jnp.maximum(nrm, 1e-12)

    return pl.pallas_call(
        body,
        grid=(n // r,),
        in_specs=[
            pl.BlockSpec((_NC, r, d), lambda i: (0, i, 0)),
            pl.BlockSpec((r, d), lambda i: (i, 0)),
            pl.BlockSpec((r, 1), lambda i: (i, 0)),
            pl.BlockSpec((d, d), lambda i: (0, 0)),
            pl.BlockSpec((d, d), lambda i: (0, 0)),
        ],
        out_specs=pl.BlockSpec((r, d), lambda i: (i, 0)),
        out_shape=jax.ShapeDtypeStruct((n, d), jnp.float32),
    )(partials, feat, deg, wt1, wt2)


def kernel(feat, edge, degree, W):
    n, d = feat.shape
    e = edge.shape[0]

    n_pad = -(-n // (_NS * 8)) * (_NS * 8)
    e_pad = -(-e // (_NW * _T)) * (_NW * _T)
    pad = e_pad - e
    src = jnp.concatenate([edge[:, 0], jnp.zeros((pad,), jnp.int32)])
    dst = jnp.concatenate([edge[:, 1], jnp.full((pad,), n, jnp.int32)])
    steps = e_pad // (_NW * _T)
    src3 = src.reshape(_NW, steps, _T)
    dst3 = dst.reshape(_NW, steps, _T)

    partials = _sc_aggregate(feat, src3, dst3, n_pad)

    wt = W.T
    deg2 = degree.astype(jnp.float32).reshape(n, 1)
    return _tc_finish(partials, feat, deg2, wt[:d], wt[d:])
